# baseline (speedup 1.0000x reference)
import functools

import jax
import jax.numpy as jnp
from jax import lax
from jax.experimental import pallas as pl
from jax.experimental.pallas import tpu as pltpu
from jax.experimental.pallas import tpu_sc as plsc

VOCAB = 1000000
D = 32
W = 20
N = 51200
NC, NS = 2, 16
NW = NC * NS
SEG_PER_W = N // NW
WI = SEG_PER_W * W
CHUNK = 64
NCHUNK = SEG_PER_W // CHUNK
CI = CHUNK * W
GROUPS = CHUNK // 16
SLICES = [(j * 128, 128) for j in range(CI // 128)]
if CI % 128:
    SLICES.append((CI - CI % 128, CI % 128))


def _tree_sum(xs):
    xs = list(xs)
    while len(xs) > 1:
        nxt = [a + b for a, b in zip(xs[0::2], xs[1::2])]
        if len(xs) % 2:
            nxt.append(xs[-1])
        xs = nxt
    return xs[0]


def _bow_body(idx_hbm, table_hbm, out_hbm,
              idx_v, rows_v0, rows_v1, out_v0, out_v1,
              sem0, sem1, osem0, osem1):
    wid = lax.axis_index("s") * NC + lax.axis_index("c")
    seg0 = wid * SEG_PER_W

    rows_b = (rows_v0, rows_v1)
    out_b = (out_v0, out_v1)
    sem_b = (sem0, sem1)
    osem_b = (osem0, osem1)

    pltpu.sync_copy(idx_hbm.at[pl.ds(seg0 * W, WI)], idx_v)

    def fire_gathers(k, b):
        for (o, s) in SLICES:
            pltpu.async_copy(
                table_hbm.at[idx_v.at[pl.ds(k * CI + o, s)]],
                rows_b[b].at[pl.ds(o, s)], sem_b[b])

    def drain_gathers(k, b):
        for (o, s) in SLICES:
            pltpu.make_async_copy(
                table_hbm.at[idx_v.at[pl.ds(k * CI + o, s)]],
                rows_b[b].at[pl.ds(o, s)], sem_b[b]).wait()

    def compute(k, b):
        rows_v, out_v = rows_b[b], out_b[b]
        zero = jnp.zeros((16,), jnp.float32)
        one = jnp.ones((16,), jnp.float32)
        kci = k * CI

        def group_body(g, _):
            base = g * 16
            for cc in range(16):
                c20 = (base + cc) * W
                m1 = jnp.where(idx_v[pl.ds(kci + c20, 16)] != 0, one, zero)
                m2 = jnp.where(idx_v[pl.ds(kci + c20 + 4, 16)] != 0,
                               one, zero)
                prods_lo = []
                prods_hi = []
                for w in range(W):
                    if w < 16:
                        m = jnp.take_along_axis(
                            m1, jnp.full((16,), w, jnp.int32), axis=0)
                    else:
                        m = jnp.take_along_axis(
                            m2, jnp.full((16,), w - 4, jnp.int32), axis=0)
                    prods_lo.append(rows_v[c20 + w, pl.ds(0, 16)] * m)
                    prods_hi.append(rows_v[c20 + w, pl.ds(16, 16)] * m)
                out_v[base + cc, pl.ds(0, 16)] = _tree_sum(prods_lo)
                out_v[base + cc, pl.ds(16, 16)] = _tree_sum(prods_hi)
            return 0

        lax.fori_loop(0, GROUPS, group_body, 0)

    def store_out(k, b):
        pltpu.async_copy(out_b[b], out_hbm.at[pl.ds(seg0 + k * CHUNK, CHUNK)],
                         osem_b[b])

    def wait_out(b):
        pltpu.make_async_copy(out_b[b], out_hbm.at[pl.ds(seg0, CHUNK)],
                              osem_b[b]).wait()

    def step(k, b, first):
        drain_gathers(k, b)
        if not first:
            wait_out(b)
        compute(k, b)

        @pl.when(k + 2 < NCHUNK)
        def _():
            fire_gathers(k + 2, b)

        store_out(k, b)

    fire_gathers(0, 0)
    fire_gathers(1, 1)

    def first_pair(_, __):
        step(0, 0, True)
        step(1, 1, True)
        return 0

    lax.fori_loop(0, 1, first_pair, 0)

    def pair_body(p, _):
        for b in range(2):
            step(2 * p + b, b, False)
        return 0

    lax.fori_loop(1, NCHUNK // 2, pair_body, 0)

    def tail(_, __):
        step(NCHUNK - 1, 0, False)
        return 0

    lax.fori_loop(0, 1, tail, 0)
    wait_out(0)
    wait_out(1)


@functools.partial(jax.jit, static_argnames=())
def _bow(idx, table):
    f = pl.kernel(
        _bow_body,
        out_type=jax.ShapeDtypeStruct((N, D), jnp.float32),
        mesh=plsc.VectorSubcoreMesh(core_axis_name="c", subcore_axis_name="s"),
        scratch_types=[
            pltpu.VMEM((WI,), jnp.int32),
            pltpu.VMEM((CI, D), jnp.float32),
            pltpu.VMEM((CI, D), jnp.float32),
            pltpu.VMEM((CHUNK, D), jnp.float32),
            pltpu.VMEM((CHUNK, D), jnp.float32),
            pltpu.SemaphoreType.DMA,
            pltpu.SemaphoreType.DMA,
            pltpu.SemaphoreType.DMA,
            pltpu.SemaphoreType.DMA,
        ],
        compiler_params=pltpu.CompilerParams(
            needs_layout_passes=False, use_tc_tiling_on_sc=False),
    )
    return f(idx, table)


def kernel(input, l, table):
    del l
    idx = input.reshape(-1)
    out = _bow(idx, table)
    return out.reshape(input.shape[0], input.shape[1], D)
